# baseline (speedup 1.0000x reference)
import functools

import jax
import jax.numpy as jnp
from jax import lax
from jax.experimental import pallas as pl
from jax.experimental.pallas import tpu as pltpu
from jax.experimental.pallas import tpu_sc as plsc

NUM_E = 8
TOPK = 2
T = 2048
D = 768
F = 3072
A = T * TOPK
BT = 256
NBLK = A // BT + NUM_E
NPAD = NBLK * BT
NMETA = 32


def _router_body(x_ref, wr_ref, idx_ref, p_ref, loss_ref):
    x = x_ref[...]
    wr = wr_ref[...]
    logits = jnp.dot(x, wr, preferred_element_type=jnp.float32)
    m = jnp.max(logits, axis=-1, keepdims=True)
    ex = jnp.exp(logits - m)
    probs = ex / jnp.sum(ex, axis=-1, keepdims=True)
    lane = jax.lax.broadcasted_iota(jnp.int32, probs.shape, 1)
    p1 = jnp.max(probs, axis=-1, keepdims=True)
    idx1 = jnp.min(jnp.where(probs == p1, lane, NUM_E), axis=-1, keepdims=True)
    sel1 = lane == idx1
    probs2 = jnp.where(sel1, -1.0, probs)
    p2 = jnp.max(probs2, axis=-1, keepdims=True)
    idx2 = jnp.min(jnp.where(probs2 == p2, lane, NUM_E), axis=-1, keepdims=True)
    sel2 = lane == idx2
    idx_ref[...] = jnp.concatenate([idx1, idx2], axis=1)
    p_ref[...] = jnp.concatenate([p1, p2], axis=1)
    cnt = sel1.astype(jnp.float32) + sel2.astype(jnp.float32)
    frac = jnp.sum(cnt, axis=0) / (float(TOPK) * float(T))
    pmean = jnp.sum(probs, axis=0) / float(T)
    loss_ref[0, 0] = float(NUM_E) * jnp.sum(frac * pmean)


def _router(x, wr):
    return pl.pallas_call(
        _router_body,
        out_shape=(
            jax.ShapeDtypeStruct((T, TOPK), jnp.int32),
            jax.ShapeDtypeStruct((T, TOPK), jnp.float32),
            jax.ShapeDtypeStruct((1, 1), jnp.float32),
        ),
        in_specs=[
            pl.BlockSpec((T, D), lambda: (0, 0)),
            pl.BlockSpec((D, NUM_E), lambda: (0, 0)),
        ],
        out_specs=(
            pl.BlockSpec((T, TOPK), lambda: (0, 0)),
            pl.BlockSpec((T, TOPK), lambda: (0, 0)),
            pl.BlockSpec(memory_space=pltpu.SMEM),
        ),
    )(x, wr)


_AW = A // 16
_NCH = _AW // 16

@functools.cache
def _make_dispatch():
    mesh = plsc.VectorSubcoreMesh(
        core_axis_name="c", subcore_axis_name="s", num_cores=1, num_subcores=16)
    return functools.partial(
        pl.kernel,
        out_type=(
            jax.ShapeDtypeStruct((NPAD, 16), jnp.int32),
            jax.ShapeDtypeStruct((A,), jnp.int32),
            jax.ShapeDtypeStruct((NMETA,), jnp.int32),
            jax.ShapeDtypeStruct((NMETA,), jnp.int32),
        ),
        mesh=mesh,
        compiler_params=pltpu.CompilerParams(needs_layout_passes=False, use_tc_tiling_on_sc=False),
        scratch_types=[
        pltpu.VMEM((_AW,), jnp.int32),
        pltpu.VMEM((_AW,), jnp.float32),
        pltpu.VMEM((16,), jnp.int32),
        pltpu.VMEM_SHARED((16, 16), jnp.int32),
        pltpu.VMEM((16, 16), jnp.int32),
        pltpu.VMEM((_AW, 16), jnp.int32),
        pltpu.VMEM((_AW,), jnp.int32),
        pltpu.VMEM((2, _AW // 2), jnp.int32),
            pltpu.VMEM((NMETA,), jnp.int32),
            pltpu.VMEM((NMETA,), jnp.int32),
            pltpu.SemaphoreType.DMA,
        ],
    )(_dispatch_body)


def _splat(s):
    return jnp.broadcast_to(s, (16,))


def _dispatch_body(idx_hbm, p_hbm, strows_hbm, pos_hbm, bexp_hbm, bvalid_hbm,
                   ids_v, p_v, cnt_v, hist_sh, hist_v, rowbuf, posl, pos2d,
                   bexp_v, bvalid_v, sem):
    w = lax.axis_index("s")
    base_a = w * _AW
    iota = lax.iota(jnp.int32, 16)
    zeros16 = jnp.zeros((16,), jnp.int32)
    ones16 = jnp.full((16,), 1, jnp.int32)

    pltpu.sync_copy(idx_hbm.at[pl.ds(base_a, _AW)], ids_v)
    pltpu.sync_copy(p_hbm.at[pl.ds(base_a, _AW)], p_v)

    acc = [jnp.zeros((16,), jnp.int32) for _ in range(NUM_E)]
    for c in range(_NCH):
        ids = ids_v[pl.ds(16 * c, 16)]
        for e in range(NUM_E):
            acc[e] = acc[e] + jnp.where(ids == e, ones16, zeros16)
    cnt_row = jnp.zeros((16,), jnp.int32)
    for e in range(NUM_E):
        cnt_row = jnp.where(iota == e, _splat(jnp.sum(acc[e])), cnt_row)
    cnt_v[...] = cnt_row
    pltpu.sync_copy(cnt_v, hist_sh.at[w])
    plsc.subcore_barrier()
    pltpu.sync_copy(hist_sh, hist_v)

    wv = _splat(w)
    starts = []
    ends = []
    base = jnp.int32(0)
    for e in range(NUM_E):
        col = plsc.load_gather(hist_v, [iota, jnp.full((16,), e, jnp.int32)])
        cum = plsc.cumsum(col)
        total = jnp.sum(jnp.where(iota == 15, cum, zeros16))
        off = jnp.sum(jnp.where(iota == wv, cum - col, zeros16))
        starts.append(base + off)
        base = base + ((total + BT - 1) // BT) * BT
        ends.append(base)

    @pl.when(w == 0)
    def _meta():
        for j in range(NMETA // 16):
            rs = (jnp.full((16,), j * 16, jnp.int32) + iota) * jnp.full(
                (16,), BT, jnp.int32)
            be = jnp.zeros((16,), jnp.int32)
            for e in range(NUM_E):
                be = be + jnp.where(rs >= _splat(ends[e]), ones16, zeros16)
            bexp_v[pl.ds(16 * j, 16)] = jnp.minimum(
                be, jnp.full((16,), NUM_E - 1, jnp.int32))
            bvalid_v[pl.ds(16 * j, 16)] = jnp.where(
                rs < _splat(ends[NUM_E - 1]), ones16, zeros16)
        pltpu.sync_copy(bexp_v, bexp_hbm)
        pltpu.sync_copy(bvalid_v, bvalid_hbm)

    run = list(starts)
    for c in range(_NCH):
        ids = ids_v[pl.ds(16 * c, 16)]
        pv = p_v[pl.ds(16 * c, 16)]
        avec = _splat(base_a + 16 * c) + iota
        tok = avec // jnp.full((16,), TOPK, jnp.int32)
        posvec = jnp.zeros((16,), jnp.int32)
        for e in range(NUM_E):
            msk = ids == e
            mi = jnp.where(msk, ones16, zeros16)
            pref = plsc.cumsum(mi)
            posvec = jnp.where(msk, _splat(run[e]) + pref - ones16, posvec)
            run[e] = run[e] + jnp.sum(mi)
        rows = jnp.full((16,), 16 * c, jnp.int32) + iota
        plsc.store_scatter(rowbuf, [rows, zeros16], tok)
        plsc.store_scatter(rowbuf, [rows, ones16], plsc.bitcast(pv, jnp.int32))
        posl[pl.ds(16 * c, 16)] = posvec
        pos2d[c // (_NCH // 2), pl.ds((c % (_NCH // 2)) * 16, 16)] = posvec

    pltpu.sync_copy(posl, pos_hbm.at[pl.ds(base_a, _AW)])
    for j in range(2):
        pltpu.async_copy(rowbuf.at[pl.ds(j * (_AW // 2), _AW // 2)],
                         strows_hbm.at[pos2d.at[j]], sem).wait()


_NW = 32
_RG = NPAD // _NW
_GC = 32
_NGC = _RG // _GC


def _both_cores_mesh():
    return plsc.VectorSubcoreMesh(
        core_axis_name="c", subcore_axis_name="s", num_cores=2, num_subcores=16)


@functools.cache
def _make_gather():
    return functools.partial(
        pl.kernel,
        out_type=(
            jax.ShapeDtypeStruct((NPAD, D), jnp.float32),
            jax.ShapeDtypeStruct((NPAD,), jnp.float32),
        ),
        mesh=_both_cores_mesh(),
        compiler_params=pltpu.CompilerParams(
            needs_layout_passes=False, use_tc_tiling_on_sc=False),
        scratch_types=[
            pltpu.VMEM((_RG, 16), jnp.int32),
            pltpu.VMEM((_NGC, _GC), jnp.int32),
            pltpu.VMEM((_RG,), jnp.float32),
            pltpu.VMEM((32,), jnp.int32),
            pltpu.VMEM((3, _GC, D), jnp.float32),
            pltpu.SemaphoreType.DMA,
            pltpu.SemaphoreType.DMA,
            pltpu.SemaphoreType.DMA,
            pltpu.SemaphoreType.DMA,
            pltpu.SemaphoreType.DMA,
            pltpu.SemaphoreType.DMA,
        ],
    )(_gather_body)


def _gather_body(x_hbm, strows_hbm, bvalid_hbm, xs_hbm, sg_hbm,
                 srt_v, tokc, sg_v, bv_v, ring, sg0, sg1, sg2, so0, so1, so2):
    gsem = (sg0, sg1, sg2)
    osem = (so0, so1, so2)
    wid = lax.axis_index("s") * 2 + lax.axis_index("c")
    base_r = wid * _RG
    iota = lax.iota(jnp.int32, 16)
    zeros16 = jnp.zeros((16,), jnp.int32)
    ones16 = jnp.full((16,), 1, jnp.int32)

    pltpu.sync_copy(strows_hbm.at[pl.ds(base_r, _RG)], srt_v)
    pltpu.sync_copy(bvalid_hbm, bv_v)
    nb = jnp.sum(bv_v[pl.ds(0, 16)]) + jnp.sum(bv_v[pl.ds(16, 16)])
    end7 = nb * BT
    for c in range(_RG // 16):
        rows = jnp.full((16,), 16 * c, jnp.int32) + iota
        tok = plsc.load_gather(srt_v, [rows, zeros16])
        tok = jnp.minimum(jnp.maximum(tok, jnp.zeros((16,), jnp.int32)),
                          jnp.full((16,), T - 1, jnp.int32))
        gb = plsc.load_gather(srt_v, [rows, ones16])
        sg_v[pl.ds(16 * c, 16)] = plsc.bitcast(gb, jnp.float32)
        h = c // (_GC // 16)
        tokc[h, pl.ds((c % (_GC // 16)) * 16, 16)] = tok
    pltpu.sync_copy(sg_v, sg_hbm.at[pl.ds(base_r, _RG)])

    def _pred(c):
        return base_r + c * _GC < end7

    def _start(c):
        pltpu.async_copy(x_hbm.at[tokc.at[c]], ring.at[c % 3], gsem[c % 3])

    @pl.when(_pred(0))
    def _p0():
        _start(0)
    for c in range(_NGC):
        if c + 1 < _NGC:
            @pl.when(_pred(c + 1))
            def _pn(c=c):
                if c + 1 >= 3:
                    pltpu.make_async_copy(
                        ring.at[(c + 1) % 3],
                        xs_hbm.at[pl.ds(base_r + (c - 2) * _GC, _GC)],
                        osem[(c + 1) % 3]).wait()
                _start(c + 1)

        @pl.when(_pred(c))
        def _pc(c=c):
            pltpu.make_async_copy(x_hbm.at[tokc.at[c]], ring.at[c % 3],
                                  gsem[c % 3]).wait()
            pltpu.async_copy(ring.at[c % 3],
                             xs_hbm.at[pl.ds(base_r + c * _GC, _GC)],
                             osem[c % 3])
    for c in range(max(0, _NGC - 3), _NGC):
        @pl.when(_pred(c))
        def _pd(c=c):
            pltpu.make_async_copy(
                ring.at[c % 3], xs_hbm.at[pl.ds(base_r + c * _GC, _GC)],
                osem[c % 3]).wait()


def _gemm_body(bexp_ref, bvalid_ref, x_ref, strows_ref, w1_ref, b1_ref,
               w2_ref, b2_ref, ys_ref):
    b = pl.program_id(0)

    @pl.when(bvalid_ref[b] != 0)
    def _():
        srows = strows_ref[...]
        st = jnp.clip(srows[:, 0:1], 0, T - 1)
        sg = jax.lax.bitcast_convert_type(srows[:, 1:2], jnp.float32)
        lane_t = jax.lax.broadcasted_iota(jnp.int32, (BT, T), 1)
        onehot = (lane_t == st).astype(jnp.bfloat16)
        xsb = jnp.dot(onehot, x_ref[...], preferred_element_type=jnp.float32)
        h = jnp.dot(xsb.astype(jnp.bfloat16), w1_ref[0].astype(jnp.bfloat16),
                    preferred_element_type=jnp.float32) + b1_ref[0]
        h = jax.nn.gelu(h)
        o = jnp.dot(h.astype(jnp.bfloat16), w2_ref[0].astype(jnp.bfloat16),
                    preferred_element_type=jnp.float32) + b2_ref[0]
        ys_ref[...] = o * sg


def _gemm(x, strows, w1, b1, w2, b2, bexp, bvalid):
    return pl.pallas_call(
        _gemm_body,
        grid_spec=pltpu.PrefetchScalarGridSpec(
            num_scalar_prefetch=2,
            grid=(NBLK,),
            in_specs=[
                pl.BlockSpec((T, D), lambda b, be, bv: (0, 0)),
                pl.BlockSpec((BT, 16), lambda b, be, bv: (b, 0)),
                pl.BlockSpec((1, D, F), lambda b, be, bv: (be[b], 0, 0)),
                pl.BlockSpec((1, 1, F), lambda b, be, bv: (be[b], 0, 0)),
                pl.BlockSpec((1, F, D), lambda b, be, bv: (be[b], 0, 0)),
                pl.BlockSpec((1, 1, D), lambda b, be, bv: (be[b], 0, 0)),
            ],
            out_specs=pl.BlockSpec((BT, D), lambda b, be, bv: (b, 0)),
        ),
        out_shape=jax.ShapeDtypeStruct((NPAD, D), jnp.float32),
    )(bexp, bvalid, x, strows, w1, b1.reshape(NUM_E, 1, F), w2,
      b2.reshape(NUM_E, 1, D))


_TW = T // _NW
_TWH = _TW // 2


@functools.cache
def _make_combine():
    return functools.partial(
        pl.kernel,
        out_type=jax.ShapeDtypeStruct((T, D), jnp.float32),
        mesh=_both_cores_mesh(),
        compiler_params=pltpu.CompilerParams(needs_layout_passes=False, use_tc_tiling_on_sc=False),
        scratch_types=[
            pltpu.VMEM((2 * _TW,), jnp.int32),
            pltpu.VMEM((2, _TWH), jnp.int32),
            pltpu.VMEM((2, _TWH), jnp.int32),
            pltpu.VMEM((_TWH, D), jnp.float32),
            pltpu.VMEM((_TWH, D), jnp.float32),
            pltpu.VMEM((_TWH, D), jnp.float32),
            pltpu.SemaphoreType.DMA,
        ],
    )(_combine_body)


def _combine_body(ys_hbm, pos_hbm, out_hbm,
                  pos_v, pe2, po2, ge_v, go_v, out_v, sem):
    wid = lax.axis_index("s") * 2 + lax.axis_index("c")
    base_t = wid * _TW
    iota = lax.iota(jnp.int32, 16)

    pltpu.sync_copy(pos_hbm.at[pl.ds(base_t * TOPK, 2 * _TW)], pos_v)
    ones16 = jnp.full((16,), 1, jnp.int32)
    for hh in range(2):
        for c in range(_TWH // 16):
            src = jnp.full((16,), hh * 2 * _TWH + 32 * c, jnp.int32) + iota + iota
            pe2[hh, pl.ds(16 * c, 16)] = plsc.load_gather(pos_v, [src])
            po2[hh, pl.ds(16 * c, 16)] = plsc.load_gather(pos_v, [src + ones16])
    for hh in range(2):
        pltpu.async_copy(ys_hbm.at[pe2.at[hh]], ge_v, sem).wait()
        pltpu.async_copy(ys_hbm.at[po2.at[hh]], go_v, sem).wait()

        def _row(i):
            for l in range(D // 16):
                out_v[i, pl.ds(16 * l, 16)] = (
                    ge_v[i, pl.ds(16 * l, 16)] + go_v[i, pl.ds(16 * l, 16)])

        pl.loop(0, _TWH)(_row)
        pltpu.sync_copy(out_v, out_hbm.at[pl.ds(base_t + hh * _TWH, _TWH)])


def kernel(input_batch, Wr, W1, b1, W2, b2):
    B, S, Dm = input_batch.shape
    x = input_batch.reshape(B * S, Dm)
    idxp, pp, loss = _router(x, Wr)
    strows, pos, bexp, bvalid = _make_dispatch()(idxp.reshape(A), pp.reshape(A))
    ys = _gemm(x.astype(jnp.bfloat16), strows, W1, b1, W2, b2, bexp, bvalid)
    out = _make_combine()(ys, pos)
    return out.reshape(B, S, Dm), loss[0, 0]


# speedup vs baseline: 1.0095x; 1.0095x over previous
import functools

import jax
import jax.numpy as jnp
from jax import lax
from jax.experimental import pallas as pl
from jax.experimental.pallas import tpu as pltpu
from jax.experimental.pallas import tpu_sc as plsc

NUM_E = 8
TOPK = 2
T = 2048
D = 768
F = 3072
A = T * TOPK
BT = 256
NBLK = A // BT + NUM_E
NPAD = NBLK * BT
NMETA = 32


def _router_body(x_ref, wr_ref, idx_ref, p_ref, loss_ref):
    x = x_ref[...]
    wr = wr_ref[...]
    logits = jnp.dot(x, wr, preferred_element_type=jnp.float32)
    m = jnp.max(logits, axis=-1, keepdims=True)
    ex = jnp.exp(logits - m)
    probs = ex / jnp.sum(ex, axis=-1, keepdims=True)
    lane = jax.lax.broadcasted_iota(jnp.int32, probs.shape, 1)
    p1 = jnp.max(probs, axis=-1, keepdims=True)
    idx1 = jnp.min(jnp.where(probs == p1, lane, NUM_E), axis=-1, keepdims=True)
    sel1 = lane == idx1
    probs2 = jnp.where(sel1, -1.0, probs)
    p2 = jnp.max(probs2, axis=-1, keepdims=True)
    idx2 = jnp.min(jnp.where(probs2 == p2, lane, NUM_E), axis=-1, keepdims=True)
    sel2 = lane == idx2
    idx_ref[...] = jnp.concatenate([idx1, idx2], axis=1)
    p_ref[...] = jnp.concatenate([p1, p2], axis=1)
    cnt = sel1.astype(jnp.float32) + sel2.astype(jnp.float32)
    frac = jnp.sum(cnt, axis=0) / (float(TOPK) * float(T))
    pmean = jnp.sum(probs, axis=0) / float(T)
    loss_ref[0, 0] = float(NUM_E) * jnp.sum(frac * pmean)


def _router(x, wr):
    return pl.pallas_call(
        _router_body,
        out_shape=(
            jax.ShapeDtypeStruct((T, TOPK), jnp.int32),
            jax.ShapeDtypeStruct((T, TOPK), jnp.float32),
            jax.ShapeDtypeStruct((1, 1), jnp.float32),
        ),
        in_specs=[
            pl.BlockSpec((T, D), lambda: (0, 0)),
            pl.BlockSpec((D, NUM_E), lambda: (0, 0)),
        ],
        out_specs=(
            pl.BlockSpec((T, TOPK), lambda: (0, 0)),
            pl.BlockSpec((T, TOPK), lambda: (0, 0)),
            pl.BlockSpec(memory_space=pltpu.SMEM),
        ),
    )(x, wr)


_AW = A // 16
_NCH = _AW // 16

@functools.cache
def _make_dispatch():
    mesh = plsc.VectorSubcoreMesh(
        core_axis_name="c", subcore_axis_name="s", num_cores=1, num_subcores=16)
    return functools.partial(
        pl.kernel,
        out_type=(
            jax.ShapeDtypeStruct((NPAD, 16), jnp.int32),
            jax.ShapeDtypeStruct((A,), jnp.int32),
            jax.ShapeDtypeStruct((NMETA,), jnp.int32),
            jax.ShapeDtypeStruct((NMETA,), jnp.int32),
        ),
        mesh=mesh,
        compiler_params=pltpu.CompilerParams(needs_layout_passes=False, use_tc_tiling_on_sc=False),
        scratch_types=[
        pltpu.VMEM((_AW,), jnp.int32),
        pltpu.VMEM((_AW,), jnp.float32),
        pltpu.VMEM((16,), jnp.int32),
        pltpu.VMEM_SHARED((16, 16), jnp.int32),
        pltpu.VMEM((16, 16), jnp.int32),
        pltpu.VMEM((_AW, 16), jnp.int32),
        pltpu.VMEM((_AW,), jnp.int32),
        pltpu.VMEM((2, _AW // 2), jnp.int32),
            pltpu.VMEM((NMETA,), jnp.int32),
            pltpu.VMEM((NMETA,), jnp.int32),
            pltpu.SemaphoreType.DMA,
        ],
    )(_dispatch_body)


def _splat(s):
    return jnp.broadcast_to(s, (16,))


def _dispatch_body(idx_hbm, p_hbm, strows_hbm, pos_hbm, bexp_hbm, bvalid_hbm,
                   ids_v, p_v, cnt_v, hist_sh, hist_v, rowbuf, posl, pos2d,
                   bexp_v, bvalid_v, sem):
    w = lax.axis_index("s")
    base_a = w * _AW
    iota = lax.iota(jnp.int32, 16)
    zeros16 = jnp.zeros((16,), jnp.int32)
    ones16 = jnp.full((16,), 1, jnp.int32)

    pltpu.sync_copy(idx_hbm.at[pl.ds(base_a, _AW)], ids_v)
    pltpu.sync_copy(p_hbm.at[pl.ds(base_a, _AW)], p_v)

    acc = [jnp.zeros((16,), jnp.int32) for _ in range(NUM_E)]
    for c in range(_NCH):
        ids = ids_v[pl.ds(16 * c, 16)]
        for e in range(NUM_E):
            acc[e] = acc[e] + jnp.where(ids == e, ones16, zeros16)
    cnt_row = jnp.zeros((16,), jnp.int32)
    for e in range(NUM_E):
        cnt_row = jnp.where(iota == e, _splat(jnp.sum(acc[e])), cnt_row)
    cnt_v[...] = cnt_row
    pltpu.sync_copy(cnt_v, hist_sh.at[w])
    plsc.subcore_barrier()
    pltpu.sync_copy(hist_sh, hist_v)

    wv = _splat(w)
    starts = []
    ends = []
    base = jnp.int32(0)
    for e in range(NUM_E):
        col = plsc.load_gather(hist_v, [iota, jnp.full((16,), e, jnp.int32)])
        cum = plsc.cumsum(col)
        total = jnp.sum(jnp.where(iota == 15, cum, zeros16))
        off = jnp.sum(jnp.where(iota == wv, cum - col, zeros16))
        starts.append(base + off)
        base = base + ((total + BT - 1) // BT) * BT
        ends.append(base)

    @pl.when(w == 0)
    def _meta():
        for j in range(NMETA // 16):
            rs = (jnp.full((16,), j * 16, jnp.int32) + iota) * jnp.full(
                (16,), BT, jnp.int32)
            be = jnp.zeros((16,), jnp.int32)
            for e in range(NUM_E):
                be = be + jnp.where(rs >= _splat(ends[e]), ones16, zeros16)
            bexp_v[pl.ds(16 * j, 16)] = jnp.minimum(
                be, jnp.full((16,), NUM_E - 1, jnp.int32))
            bvalid_v[pl.ds(16 * j, 16)] = jnp.where(
                rs < _splat(ends[NUM_E - 1]), ones16, zeros16)
        pltpu.sync_copy(bexp_v, bexp_hbm)
        pltpu.sync_copy(bvalid_v, bvalid_hbm)

    run = list(starts)
    for c in range(_NCH):
        ids = ids_v[pl.ds(16 * c, 16)]
        pv = p_v[pl.ds(16 * c, 16)]
        avec = _splat(base_a + 16 * c) + iota
        tok = avec // jnp.full((16,), TOPK, jnp.int32)
        posvec = jnp.zeros((16,), jnp.int32)
        for e in range(NUM_E):
            msk = ids == e
            mi = jnp.where(msk, ones16, zeros16)
            pref = plsc.cumsum(mi)
            posvec = jnp.where(msk, _splat(run[e]) + pref - ones16, posvec)
            run[e] = run[e] + jnp.sum(mi)
        rows = jnp.full((16,), 16 * c, jnp.int32) + iota
        plsc.store_scatter(rowbuf, [rows, zeros16], tok)
        plsc.store_scatter(rowbuf, [rows, ones16], plsc.bitcast(pv, jnp.int32))
        posl[pl.ds(16 * c, 16)] = posvec
        pos2d[c // (_NCH // 2), pl.ds((c % (_NCH // 2)) * 16, 16)] = posvec

    pltpu.sync_copy(posl, pos_hbm.at[pl.ds(base_a, _AW)])
    for j in range(2):
        pltpu.async_copy(rowbuf.at[pl.ds(j * (_AW // 2), _AW // 2)],
                         strows_hbm.at[pos2d.at[j]], sem).wait()


_NW = 32
_RG = NPAD // _NW
_GC = 32
_NGC = _RG // _GC


def _both_cores_mesh():
    return plsc.VectorSubcoreMesh(
        core_axis_name="c", subcore_axis_name="s", num_cores=2, num_subcores=16)


@functools.cache
def _make_gather():
    return functools.partial(
        pl.kernel,
        out_type=(
            jax.ShapeDtypeStruct((NPAD, D), jnp.float32),
            jax.ShapeDtypeStruct((NPAD,), jnp.float32),
        ),
        mesh=_both_cores_mesh(),
        compiler_params=pltpu.CompilerParams(
            needs_layout_passes=False, use_tc_tiling_on_sc=False),
        scratch_types=[
            pltpu.VMEM((_RG, 16), jnp.int32),
            pltpu.VMEM((_NGC, _GC), jnp.int32),
            pltpu.VMEM((_RG,), jnp.float32),
            pltpu.VMEM((32,), jnp.int32),
            pltpu.VMEM((3, _GC, D), jnp.float32),
            pltpu.SemaphoreType.DMA,
            pltpu.SemaphoreType.DMA,
            pltpu.SemaphoreType.DMA,
            pltpu.SemaphoreType.DMA,
            pltpu.SemaphoreType.DMA,
            pltpu.SemaphoreType.DMA,
        ],
    )(_gather_body)


def _gather_body(x_hbm, strows_hbm, bvalid_hbm, xs_hbm, sg_hbm,
                 srt_v, tokc, sg_v, bv_v, ring, sg0, sg1, sg2, so0, so1, so2):
    gsem = (sg0, sg1, sg2)
    osem = (so0, so1, so2)
    wid = lax.axis_index("s") * 2 + lax.axis_index("c")
    base_r = wid * _RG
    iota = lax.iota(jnp.int32, 16)
    zeros16 = jnp.zeros((16,), jnp.int32)
    ones16 = jnp.full((16,), 1, jnp.int32)

    pltpu.sync_copy(strows_hbm.at[pl.ds(base_r, _RG)], srt_v)
    pltpu.sync_copy(bvalid_hbm, bv_v)
    nb = jnp.sum(bv_v[pl.ds(0, 16)]) + jnp.sum(bv_v[pl.ds(16, 16)])
    end7 = nb * BT
    for c in range(_RG // 16):
        rows = jnp.full((16,), 16 * c, jnp.int32) + iota
        tok = plsc.load_gather(srt_v, [rows, zeros16])
        tok = jnp.minimum(jnp.maximum(tok, jnp.zeros((16,), jnp.int32)),
                          jnp.full((16,), T - 1, jnp.int32))
        gb = plsc.load_gather(srt_v, [rows, ones16])
        sg_v[pl.ds(16 * c, 16)] = plsc.bitcast(gb, jnp.float32)
        h = c // (_GC // 16)
        tokc[h, pl.ds((c % (_GC // 16)) * 16, 16)] = tok
    pltpu.sync_copy(sg_v, sg_hbm.at[pl.ds(base_r, _RG)])

    def _pred(c):
        return base_r + c * _GC < end7

    def _start(c):
        pltpu.async_copy(x_hbm.at[tokc.at[c]], ring.at[c % 3], gsem[c % 3])

    @pl.when(_pred(0))
    def _p0():
        _start(0)
    for c in range(_NGC):
        if c + 1 < _NGC:
            @pl.when(_pred(c + 1))
            def _pn(c=c):
                if c + 1 >= 3:
                    pltpu.make_async_copy(
                        ring.at[(c + 1) % 3],
                        xs_hbm.at[pl.ds(base_r + (c - 2) * _GC, _GC)],
                        osem[(c + 1) % 3]).wait()
                _start(c + 1)

        @pl.when(_pred(c))
        def _pc(c=c):
            pltpu.make_async_copy(x_hbm.at[tokc.at[c]], ring.at[c % 3],
                                  gsem[c % 3]).wait()
            pltpu.async_copy(ring.at[c % 3],
                             xs_hbm.at[pl.ds(base_r + c * _GC, _GC)],
                             osem[c % 3])
    for c in range(max(0, _NGC - 3), _NGC):
        @pl.when(_pred(c))
        def _pd(c=c):
            pltpu.make_async_copy(
                ring.at[c % 3], xs_hbm.at[pl.ds(base_r + c * _GC, _GC)],
                osem[c % 3]).wait()


def _gemm_body(bexp_ref, bvalid_ref, x_ref, strows_ref, w1_ref, b1_ref,
               w2_ref, b2_ref, ys_ref):
    b = pl.program_id(0)

    @pl.when(bvalid_ref[b] != 0)
    def _():
        srows = strows_ref[...]
        st = jnp.clip(srows[:, 0:1], 0, T - 1)
        sg = jax.lax.bitcast_convert_type(srows[:, 1:2], jnp.float32)
        lane_t = jax.lax.broadcasted_iota(jnp.int32, (BT, T), 1)
        onehot = (lane_t == st).astype(jnp.float32)
        xsb = jnp.dot(onehot, x_ref[...], preferred_element_type=jnp.float32)
        h = jnp.dot(xsb, w1_ref[0],
                    preferred_element_type=jnp.float32) + b1_ref[0]
        h = jax.nn.gelu(h)
        o = jnp.dot(h, w2_ref[0], preferred_element_type=jnp.float32) + b2_ref[0]
        ys_ref[...] = o * sg


def _gemm(x, strows, w1, b1, w2, b2, bexp, bvalid):
    return pl.pallas_call(
        _gemm_body,
        grid_spec=pltpu.PrefetchScalarGridSpec(
            num_scalar_prefetch=2,
            grid=(NBLK,),
            in_specs=[
                pl.BlockSpec((T, D), lambda b, be, bv: (0, 0)),
                pl.BlockSpec((BT, 16), lambda b, be, bv: (b, 0)),
                pl.BlockSpec((1, D, F), lambda b, be, bv: (be[b], 0, 0)),
                pl.BlockSpec((1, 1, F), lambda b, be, bv: (be[b], 0, 0)),
                pl.BlockSpec((1, F, D), lambda b, be, bv: (be[b], 0, 0)),
                pl.BlockSpec((1, 1, D), lambda b, be, bv: (be[b], 0, 0)),
            ],
            out_specs=pl.BlockSpec((BT, D), lambda b, be, bv: (b, 0)),
        ),
        out_shape=jax.ShapeDtypeStruct((NPAD, D), jnp.float32),
        compiler_params=pltpu.CompilerParams(
            vmem_limit_bytes=100 * 1024 * 1024),
    )(bexp, bvalid, x, strows, w1, b1.reshape(NUM_E, 1, F), w2,
      b2.reshape(NUM_E, 1, D))


_TW = T // _NW
_TWH = _TW // 2


@functools.cache
def _make_combine():
    return functools.partial(
        pl.kernel,
        out_type=jax.ShapeDtypeStruct((T, D), jnp.float32),
        mesh=_both_cores_mesh(),
        compiler_params=pltpu.CompilerParams(needs_layout_passes=False, use_tc_tiling_on_sc=False),
        scratch_types=[
            pltpu.VMEM((2 * _TW,), jnp.int32),
            pltpu.VMEM((2, _TWH), jnp.int32),
            pltpu.VMEM((2, _TWH), jnp.int32),
            pltpu.VMEM((_TWH, D), jnp.float32),
            pltpu.VMEM((_TWH, D), jnp.float32),
            pltpu.VMEM((_TWH, D), jnp.float32),
            pltpu.SemaphoreType.DMA,
        ],
    )(_combine_body)


def _combine_body(ys_hbm, pos_hbm, out_hbm,
                  pos_v, pe2, po2, ge_v, go_v, out_v, sem):
    wid = lax.axis_index("s") * 2 + lax.axis_index("c")
    base_t = wid * _TW
    iota = lax.iota(jnp.int32, 16)

    pltpu.sync_copy(pos_hbm.at[pl.ds(base_t * TOPK, 2 * _TW)], pos_v)
    ones16 = jnp.full((16,), 1, jnp.int32)
    for hh in range(2):
        for c in range(_TWH // 16):
            src = jnp.full((16,), hh * 2 * _TWH + 32 * c, jnp.int32) + iota + iota
            pe2[hh, pl.ds(16 * c, 16)] = plsc.load_gather(pos_v, [src])
            po2[hh, pl.ds(16 * c, 16)] = plsc.load_gather(pos_v, [src + ones16])
    for hh in range(2):
        pltpu.async_copy(ys_hbm.at[pe2.at[hh]], ge_v, sem).wait()
        pltpu.async_copy(ys_hbm.at[po2.at[hh]], go_v, sem).wait()

        def _row(i):
            for l in range(D // 16):
                out_v[i, pl.ds(16 * l, 16)] = (
                    ge_v[i, pl.ds(16 * l, 16)] + go_v[i, pl.ds(16 * l, 16)])

        pl.loop(0, _TWH)(_row)
        pltpu.sync_copy(out_v, out_hbm.at[pl.ds(base_t + hh * _TWH, _TWH)])


def kernel(input_batch, Wr, W1, b1, W2, b2):
    B, S, Dm = input_batch.shape
    x = input_batch.reshape(B * S, Dm)
    idxp, pp, loss = _router(x, Wr)
    strows, pos, bexp, bvalid = _make_dispatch()(idxp.reshape(A), pp.reshape(A))
    ys = _gemm(x, strows, W1, b1, W2, b2, bexp, bvalid)
    out = _make_combine()(ys, pos)
    return out.reshape(B, S, Dm), loss[0, 0]


# speedup vs baseline: 1.0218x; 1.0122x over previous
import functools

import jax
import jax.numpy as jnp
from jax import lax
from jax.experimental import pallas as pl
from jax.experimental.pallas import tpu as pltpu
from jax.experimental.pallas import tpu_sc as plsc

NUM_E = 8
TOPK = 2
T = 2048
D = 768
F = 3072
A = T * TOPK
BT = 256
NBLK = A // BT + NUM_E
NPAD = NBLK * BT
NMETA = 32


def _router_body(x_ref, wr_ref, idx_ref, p_ref, loss_ref):
    x = x_ref[...]
    wr = wr_ref[...]
    logits = jnp.dot(x, wr, preferred_element_type=jnp.float32)
    m = jnp.max(logits, axis=-1, keepdims=True)
    ex = jnp.exp(logits - m)
    probs = ex / jnp.sum(ex, axis=-1, keepdims=True)
    lane = jax.lax.broadcasted_iota(jnp.int32, probs.shape, 1)
    p1 = jnp.max(probs, axis=-1, keepdims=True)
    idx1 = jnp.min(jnp.where(probs == p1, lane, NUM_E), axis=-1, keepdims=True)
    sel1 = lane == idx1
    probs2 = jnp.where(sel1, -1.0, probs)
    p2 = jnp.max(probs2, axis=-1, keepdims=True)
    idx2 = jnp.min(jnp.where(probs2 == p2, lane, NUM_E), axis=-1, keepdims=True)
    sel2 = lane == idx2
    idx_ref[...] = jnp.concatenate([idx1, idx2], axis=1)
    p_ref[...] = jnp.concatenate([p1, p2], axis=1)
    cnt = sel1.astype(jnp.float32) + sel2.astype(jnp.float32)
    frac = jnp.sum(cnt, axis=0) / (float(TOPK) * float(T))
    pmean = jnp.sum(probs, axis=0) / float(T)
    loss_ref[0, 0] = float(NUM_E) * jnp.sum(frac * pmean)


def _router(x, wr):
    return pl.pallas_call(
        _router_body,
        out_shape=(
            jax.ShapeDtypeStruct((T, TOPK), jnp.int32),
            jax.ShapeDtypeStruct((T, TOPK), jnp.float32),
            jax.ShapeDtypeStruct((1, 1), jnp.float32),
        ),
        in_specs=[
            pl.BlockSpec((T, D), lambda: (0, 0)),
            pl.BlockSpec((D, NUM_E), lambda: (0, 0)),
        ],
        out_specs=(
            pl.BlockSpec((T, TOPK), lambda: (0, 0)),
            pl.BlockSpec((T, TOPK), lambda: (0, 0)),
            pl.BlockSpec(memory_space=pltpu.SMEM),
        ),
    )(x, wr)


_AW = A // 16
_NCH = _AW // 16

@functools.cache
def _make_dispatch():
    mesh = plsc.VectorSubcoreMesh(
        core_axis_name="c", subcore_axis_name="s", num_cores=1, num_subcores=16)
    return functools.partial(
        pl.kernel,
        out_type=(
            jax.ShapeDtypeStruct((NPAD, 16), jnp.int32),
            jax.ShapeDtypeStruct((A,), jnp.int32),
            jax.ShapeDtypeStruct((NMETA,), jnp.int32),
            jax.ShapeDtypeStruct((NMETA,), jnp.int32),
        ),
        mesh=mesh,
        compiler_params=pltpu.CompilerParams(needs_layout_passes=False, use_tc_tiling_on_sc=False),
        scratch_types=[
        pltpu.VMEM((_AW,), jnp.int32),
        pltpu.VMEM((_AW,), jnp.float32),
        pltpu.VMEM((16,), jnp.int32),
        pltpu.VMEM_SHARED((16, 16), jnp.int32),
        pltpu.VMEM((16, 16), jnp.int32),
        pltpu.VMEM((_AW, 16), jnp.int32),
        pltpu.VMEM((_AW,), jnp.int32),
        pltpu.VMEM((2, _AW // 2), jnp.int32),
            pltpu.VMEM((NMETA,), jnp.int32),
            pltpu.VMEM((NMETA,), jnp.int32),
            pltpu.SemaphoreType.DMA,
        ],
    )(_dispatch_body)


def _splat(s):
    return jnp.broadcast_to(s, (16,))


def _dispatch_body(idx_hbm, p_hbm, strows_hbm, pos_hbm, bexp_hbm, bvalid_hbm,
                   ids_v, p_v, cnt_v, hist_sh, hist_v, rowbuf, posl, pos2d,
                   bexp_v, bvalid_v, sem):
    w = lax.axis_index("s")
    base_a = w * _AW
    iota = lax.iota(jnp.int32, 16)
    zeros16 = jnp.zeros((16,), jnp.int32)
    ones16 = jnp.full((16,), 1, jnp.int32)

    pltpu.sync_copy(idx_hbm.at[pl.ds(base_a, _AW)], ids_v)
    pltpu.sync_copy(p_hbm.at[pl.ds(base_a, _AW)], p_v)

    acc = [jnp.zeros((16,), jnp.int32) for _ in range(NUM_E)]
    for c in range(_NCH):
        ids = ids_v[pl.ds(16 * c, 16)]
        for e in range(NUM_E):
            acc[e] = acc[e] + jnp.where(ids == e, ones16, zeros16)
    cnt_row = jnp.zeros((16,), jnp.int32)
    for e in range(NUM_E):
        cnt_row = jnp.where(iota == e, _splat(jnp.sum(acc[e])), cnt_row)
    cnt_v[...] = cnt_row
    pltpu.sync_copy(cnt_v, hist_sh.at[w])
    plsc.subcore_barrier()
    pltpu.sync_copy(hist_sh, hist_v)

    wv = _splat(w)
    starts = []
    ends = []
    base = jnp.int32(0)
    for e in range(NUM_E):
        col = plsc.load_gather(hist_v, [iota, jnp.full((16,), e, jnp.int32)])
        cum = plsc.cumsum(col)
        total = jnp.sum(jnp.where(iota == 15, cum, zeros16))
        off = jnp.sum(jnp.where(iota == wv, cum - col, zeros16))
        starts.append(base + off)
        base = base + ((total + BT - 1) // BT) * BT
        ends.append(base)

    @pl.when(w == 0)
    def _meta():
        for j in range(NMETA // 16):
            rs = (jnp.full((16,), j * 16, jnp.int32) + iota) * jnp.full(
                (16,), BT, jnp.int32)
            be = jnp.zeros((16,), jnp.int32)
            for e in range(NUM_E):
                be = be + jnp.where(rs >= _splat(ends[e]), ones16, zeros16)
            bexp_v[pl.ds(16 * j, 16)] = jnp.minimum(
                be, jnp.full((16,), NUM_E - 1, jnp.int32))
            bvalid_v[pl.ds(16 * j, 16)] = jnp.where(
                rs < _splat(ends[NUM_E - 1]), ones16, zeros16)
        pltpu.sync_copy(bexp_v, bexp_hbm)
        pltpu.sync_copy(bvalid_v, bvalid_hbm)

    run = list(starts)
    for c in range(_NCH):
        ids = ids_v[pl.ds(16 * c, 16)]
        pv = p_v[pl.ds(16 * c, 16)]
        avec = _splat(base_a + 16 * c) + iota
        tok = avec // jnp.full((16,), TOPK, jnp.int32)
        posvec = jnp.zeros((16,), jnp.int32)
        for e in range(NUM_E):
            msk = ids == e
            mi = jnp.where(msk, ones16, zeros16)
            pref = plsc.cumsum(mi)
            posvec = jnp.where(msk, _splat(run[e]) + pref - ones16, posvec)
            run[e] = run[e] + jnp.sum(mi)
        rows = jnp.full((16,), 16 * c, jnp.int32) + iota
        plsc.store_scatter(rowbuf, [rows, zeros16], tok)
        plsc.store_scatter(rowbuf, [rows, ones16], plsc.bitcast(pv, jnp.int32))
        posl[pl.ds(16 * c, 16)] = posvec
        pos2d[c // (_NCH // 2), pl.ds((c % (_NCH // 2)) * 16, 16)] = posvec

    pltpu.sync_copy(posl, pos_hbm.at[pl.ds(base_a, _AW)])
    for j in range(2):
        pltpu.async_copy(rowbuf.at[pl.ds(j * (_AW // 2), _AW // 2)],
                         strows_hbm.at[pos2d.at[j]], sem).wait()


_NW = 32
_RG = NPAD // _NW
_GC = 32
_NGC = _RG // _GC


def _both_cores_mesh():
    return plsc.VectorSubcoreMesh(
        core_axis_name="c", subcore_axis_name="s", num_cores=2, num_subcores=16)


@functools.cache
def _make_gather():
    return functools.partial(
        pl.kernel,
        out_type=(
            jax.ShapeDtypeStruct((NPAD, D), jnp.float32),
            jax.ShapeDtypeStruct((NPAD,), jnp.float32),
        ),
        mesh=_both_cores_mesh(),
        compiler_params=pltpu.CompilerParams(
            needs_layout_passes=False, use_tc_tiling_on_sc=False),
        scratch_types=[
            pltpu.VMEM((_RG, 16), jnp.int32),
            pltpu.VMEM((_NGC, _GC), jnp.int32),
            pltpu.VMEM((_RG,), jnp.float32),
            pltpu.VMEM((32,), jnp.int32),
            pltpu.VMEM((3, _GC, D), jnp.float32),
            pltpu.SemaphoreType.DMA,
            pltpu.SemaphoreType.DMA,
            pltpu.SemaphoreType.DMA,
            pltpu.SemaphoreType.DMA,
            pltpu.SemaphoreType.DMA,
            pltpu.SemaphoreType.DMA,
        ],
    )(_gather_body)


def _gather_body(x_hbm, strows_hbm, bvalid_hbm, xs_hbm, sg_hbm,
                 srt_v, tokc, sg_v, bv_v, ring, sg0, sg1, sg2, so0, so1, so2):
    gsem = (sg0, sg1, sg2)
    osem = (so0, so1, so2)
    wid = lax.axis_index("s") * 2 + lax.axis_index("c")
    base_r = wid * _RG
    iota = lax.iota(jnp.int32, 16)
    zeros16 = jnp.zeros((16,), jnp.int32)
    ones16 = jnp.full((16,), 1, jnp.int32)

    pltpu.sync_copy(strows_hbm.at[pl.ds(base_r, _RG)], srt_v)
    pltpu.sync_copy(bvalid_hbm, bv_v)
    nb = jnp.sum(bv_v[pl.ds(0, 16)]) + jnp.sum(bv_v[pl.ds(16, 16)])
    end7 = nb * BT
    for c in range(_RG // 16):
        rows = jnp.full((16,), 16 * c, jnp.int32) + iota
        tok = plsc.load_gather(srt_v, [rows, zeros16])
        tok = jnp.minimum(jnp.maximum(tok, jnp.zeros((16,), jnp.int32)),
                          jnp.full((16,), T - 1, jnp.int32))
        gb = plsc.load_gather(srt_v, [rows, ones16])
        sg_v[pl.ds(16 * c, 16)] = plsc.bitcast(gb, jnp.float32)
        h = c // (_GC // 16)
        tokc[h, pl.ds((c % (_GC // 16)) * 16, 16)] = tok
    pltpu.sync_copy(sg_v, sg_hbm.at[pl.ds(base_r, _RG)])

    def _pred(c):
        return base_r + c * _GC < end7

    def _start(c):
        pltpu.async_copy(x_hbm.at[tokc.at[c]], ring.at[c % 3], gsem[c % 3])

    @pl.when(_pred(0))
    def _p0():
        _start(0)
    for c in range(_NGC):
        if c + 1 < _NGC:
            @pl.when(_pred(c + 1))
            def _pn(c=c):
                if c + 1 >= 3:
                    pltpu.make_async_copy(
                        ring.at[(c + 1) % 3],
                        xs_hbm.at[pl.ds(base_r + (c - 2) * _GC, _GC)],
                        osem[(c + 1) % 3]).wait()
                _start(c + 1)

        @pl.when(_pred(c))
        def _pc(c=c):
            pltpu.make_async_copy(x_hbm.at[tokc.at[c]], ring.at[c % 3],
                                  gsem[c % 3]).wait()
            pltpu.async_copy(ring.at[c % 3],
                             xs_hbm.at[pl.ds(base_r + c * _GC, _GC)],
                             osem[c % 3])
    for c in range(max(0, _NGC - 3), _NGC):
        @pl.when(_pred(c))
        def _pd(c=c):
            pltpu.make_async_copy(
                ring.at[c % 3], xs_hbm.at[pl.ds(base_r + c * _GC, _GC)],
                osem[c % 3]).wait()


def _gemm_body(bexp_ref, bvalid_ref, x_ref, strows_ref, w1_ref, b1_ref,
               w2_ref, b2_ref, ys_ref):
    b = pl.program_id(0)

    @pl.when(bvalid_ref[b] != 0)
    def _():
        srows = strows_ref[...]
        st = jnp.clip(srows[:, 0:1], 0, T - 1)
        sg = jax.lax.bitcast_convert_type(srows[:, 1:2], jnp.float32)
        lane_t = jax.lax.broadcasted_iota(jnp.int32, (BT, T), 1)
        onehot = (lane_t == st).astype(jnp.float32)
        xsb = jnp.dot(onehot, x_ref[...], preferred_element_type=jnp.float32)
        h = jnp.dot(xsb, w1_ref[0],
                    preferred_element_type=jnp.float32) + b1_ref[0]
        h = jax.nn.gelu(h)
        o = jnp.dot(h, w2_ref[0], preferred_element_type=jnp.float32) + b2_ref[0]
        ys_ref[...] = o * sg


def _gemm(x, strows, w1, b1, w2, b2, bexp, bvalid):
    return pl.pallas_call(
        _gemm_body,
        grid_spec=pltpu.PrefetchScalarGridSpec(
            num_scalar_prefetch=2,
            grid=(NBLK,),
            in_specs=[
                pl.BlockSpec((T, D), lambda b, be, bv: (0, 0)),
                pl.BlockSpec((BT, 16), lambda b, be, bv: (b, 0)),
                pl.BlockSpec((1, D, F), lambda b, be, bv: (be[b], 0, 0)),
                pl.BlockSpec((1, 1, F), lambda b, be, bv: (be[b], 0, 0)),
                pl.BlockSpec((1, F, D), lambda b, be, bv: (be[b], 0, 0)),
                pl.BlockSpec((1, 1, D), lambda b, be, bv: (be[b], 0, 0)),
            ],
            out_specs=pl.BlockSpec((BT, D), lambda b, be, bv: (b, 0)),
        ),
        out_shape=jax.ShapeDtypeStruct((NPAD, D), jnp.float32),
        compiler_params=pltpu.CompilerParams(
            vmem_limit_bytes=100 * 1024 * 1024),
    )(bexp, bvalid, x, strows, w1, b1.reshape(NUM_E, 1, F), w2,
      b2.reshape(NUM_E, 1, D))


_TW = T // _NW
_TWH = _TW // 2


@functools.cache
def _make_combine():
    return functools.partial(
        pl.kernel,
        out_type=jax.ShapeDtypeStruct((T, D), jnp.float32),
        mesh=_both_cores_mesh(),
        compiler_params=pltpu.CompilerParams(
            needs_layout_passes=False, use_tc_tiling_on_sc=False),
        scratch_types=[
            pltpu.VMEM((2 * _TW,), jnp.int32),
            pltpu.VMEM((2, _TWH), jnp.int32),
            pltpu.VMEM((2, _TWH), jnp.int32),
            pltpu.VMEM((2, _TWH, D), jnp.float32),
            pltpu.VMEM((2, _TWH, D), jnp.float32),
            pltpu.VMEM((_TWH, D), jnp.float32),
            pltpu.SemaphoreType.DMA,
            pltpu.SemaphoreType.DMA,
            pltpu.SemaphoreType.DMA,
            pltpu.SemaphoreType.DMA,
            pltpu.SemaphoreType.DMA,
        ],
    )(_combine_body)


def _combine_body(ys_hbm, pos_hbm, out_hbm,
                  pos_v, pe2, po2, ge_r, go_r, out_v, se0, se1, so0, so1, sc):
    wid = lax.axis_index("s") * 2 + lax.axis_index("c")
    base_t = wid * _TW
    iota = lax.iota(jnp.int32, 16)

    pltpu.sync_copy(pos_hbm.at[pl.ds(base_t * TOPK, 2 * _TW)], pos_v)
    ones16 = jnp.full((16,), 1, jnp.int32)
    for hh in range(2):
        for c in range(_TWH // 16):
            src = jnp.full((16,), hh * 2 * _TWH + 32 * c, jnp.int32) + iota + iota
            pe2[hh, pl.ds(16 * c, 16)] = plsc.load_gather(pos_v, [src])
            po2[hh, pl.ds(16 * c, 16)] = plsc.load_gather(pos_v, [src + ones16])
    pltpu.async_copy(ys_hbm.at[pe2.at[0]], ge_r.at[0], se0)
    pltpu.async_copy(ys_hbm.at[po2.at[0]], go_r.at[0], so0)
    pltpu.async_copy(ys_hbm.at[pe2.at[1]], ge_r.at[1], se1)
    pltpu.async_copy(ys_hbm.at[po2.at[1]], go_r.at[1], so1)
    sems = ((se0, so0), (se1, so1))
    for hh in range(2):
        pltpu.make_async_copy(ys_hbm.at[pe2.at[hh]], ge_r.at[hh],
                              sems[hh][0]).wait()
        pltpu.make_async_copy(ys_hbm.at[po2.at[hh]], go_r.at[hh],
                              sems[hh][1]).wait()
        if hh == 1:
            pltpu.make_async_copy(out_v, out_hbm.at[pl.ds(base_t, _TWH)],
                                  sc).wait()

        def _row(i, hh=hh):
            for l in range(D // 16):
                out_v[i, pl.ds(16 * l, 16)] = (
                    ge_r[hh, i, pl.ds(16 * l, 16)]
                    + go_r[hh, i, pl.ds(16 * l, 16)])

        pl.loop(0, _TWH)(_row)
        if hh == 0:
            pltpu.async_copy(out_v, out_hbm.at[pl.ds(base_t, _TWH)], sc)
        else:
            pltpu.sync_copy(out_v, out_hbm.at[pl.ds(base_t + _TWH, _TWH)])


def kernel(input_batch, Wr, W1, b1, W2, b2):
    B, S, Dm = input_batch.shape
    x = input_batch.reshape(B * S, Dm)
    idxp, pp, loss = _router(x, Wr)
    strows, pos, bexp, bvalid = _make_dispatch()(idxp.reshape(A), pp.reshape(A))
    ys = _gemm(x, strows, W1, b1, W2, b2, bexp, bvalid)
    out = _make_combine()(ys, pos)
    return out.reshape(B, S, Dm), loss[0, 0]
